# baseline (speedup 1.0000x reference)
import functools

import jax
import jax.numpy as jnp
from jax import lax
from jax.experimental import pallas as pl
from jax.experimental.pallas import tpu as pltpu
from jax.experimental.pallas import tpu_sc as plsc

TM = 256
TK = 512
WIN = 2736
NC = 2
NS = 16
CH = 64


def _merge(acc, v, i):
    av, ai = acc
    keep = (av < v) | ((av == v) & (ai <= i))
    return jnp.where(keep, av, v), jnp.where(keep, ai, i)


def _bins_body(a_ref, ct_ref, a2_ref, c2_ref, out_ref):
    k = ct_ref.shape[1]
    windows = [(0, WIN), (WIN, 2 * WIN), (2 * WIN, k)]
    a = a_ref[...]
    a2 = a2_ref[...]
    m = a.shape[0]
    accs = [(jnp.full((m, 1), jnp.inf, jnp.float32),
             jnp.full((m, 1), k, jnp.int32)) for _ in windows]
    for j in range(k // TK):
        lo = j * TK
        p = lax.dot_general(a, ct_ref[:, lo:lo + TK], (((1,), (0,)), ((), ())),
                            preferred_element_type=jnp.float32)
        d2 = a2 - 2.0 * p + c2_ref[:, lo:lo + TK]
        dist = jnp.sqrt(jnp.maximum(d2, 0.0))
        ii = lax.broadcasted_iota(jnp.int32, dist.shape, 1) + lo
        for w, (wlo, whi) in enumerate(windows):
            if whi <= lo or wlo >= lo + TK:
                continue
            if wlo <= lo and whi >= lo + TK:
                dm = dist
            else:
                inwin = (ii >= wlo) & (ii < whi)
                dm = jnp.where(inwin, dist, jnp.inf)
            v = jnp.min(dm, axis=1, keepdims=True)
            i = jnp.min(jnp.where(dm == v, ii, k), axis=1, keepdims=True)
            accs[w] = _merge(accs[w], v, i)
    av, ai = accs[0]
    for w in (1, 2):
        av = av.astype(jnp.bfloat16).astype(jnp.float32)
        av, ai = _merge((av, ai), accs[w][0], accs[w][1])
    out_ref[...] = ai


def _compute_bins(a_bf, ct_bf, a2, c2):
    n, d = a_bf.shape
    k = ct_bf.shape[1]
    return pl.pallas_call(
        _bins_body,
        grid=(n // TM,),
        in_specs=[
            pl.BlockSpec((TM, d), lambda i: (i, 0)),
            pl.BlockSpec((d, k), lambda i: (0, 0)),
            pl.BlockSpec((TM, 1), lambda i: (i, 0)),
            pl.BlockSpec((1, k), lambda i: (0, 0)),
        ],
        out_specs=pl.BlockSpec((TM, 1), lambda i: (i, 0)),
        out_shape=jax.ShapeDtypeStruct((n, 1), jnp.int32),
    )(a_bf, ct_bf, a2, c2)


def _residual_body(n, cent_hbm, bins_hbm, act_hbm, out_hbm,
                   idx_v, rows_v, act_v, sem):
    wid = lax.axis_index("s") * NC + lax.axis_index("c")
    per_w = n // (NC * NS)
    d = act_v.shape[1]

    def chunk(t, carry):
        base = wid * per_w + t * CH
        pltpu.sync_copy(bins_hbm.at[pl.ds(base, CH)], idx_v)
        gather = pltpu.async_copy(cent_hbm.at[idx_v], rows_v, sem)
        pltpu.sync_copy(act_hbm.at[pl.ds(base, CH), :], act_v)
        gather.wait()

        def sub(i, c):
            r = i // (d // 16)
            o = (i % (d // 16)) * 16
            act_v[r, pl.ds(o, 16)] = (act_v[r, pl.ds(o, 16)]
                                      - rows_v[r, pl.ds(o, 16)])
            return c

        lax.fori_loop(0, CH * (d // 16), sub, 0)
        pltpu.sync_copy(act_v, out_hbm.at[pl.ds(base, CH), :])
        return carry

    lax.fori_loop(0, per_w // CH, chunk, 0)


def _compute_residual(centroids, bins, flat):
    n, d = flat.shape
    mesh = plsc.VectorSubcoreMesh(core_axis_name="c", subcore_axis_name="s")
    fn = functools.partial(
        pl.kernel,
        out_type=jax.ShapeDtypeStruct((n, d), jnp.float32),
        mesh=mesh,
        scratch_types=[
            pltpu.VMEM((CH,), jnp.int32),
            pltpu.VMEM((CH, d), jnp.float32),
            pltpu.VMEM((CH, d), jnp.float32),
            pltpu.SemaphoreType.DMA,
        ],
    )(functools.partial(_residual_body, n))
    return fn(centroids, bins, flat)


def kernel(action, centroids):
    b, t, d = action.shape
    k = centroids.shape[0]
    flat = action.reshape(b * t, d)
    a2 = jnp.sum(flat * flat, axis=1)
    c2 = jnp.sum(centroids * centroids, axis=1)
    a_bf = flat.astype(jnp.bfloat16)
    ct_bf = centroids.T.astype(jnp.bfloat16)
    bins2d = _compute_bins(a_bf, ct_bf, a2.reshape(b * t, 1),
                           c2.reshape(1, k))
    residual = _compute_residual(centroids, bins2d.reshape(b * t), flat)
    return (bins2d.reshape(b, t, 1).astype(jnp.int64),
            residual.reshape(b, t, d))


# speedup vs baseline: 1.0480x; 1.0480x over previous
import functools

import jax
import jax.numpy as jnp
from jax import lax
from jax.experimental import pallas as pl
from jax.experimental.pallas import tpu as pltpu
from jax.experimental.pallas import tpu_sc as plsc

TM = 64
TK = 512
SL = 128
WIN = 2736
NC = 2
NS = 16
CH = 64


def _merge(acc, v, i):
    av, ai = acc
    keep = (av < v) | ((av == v) & (ai <= i))
    return jnp.where(keep, av, v), jnp.where(keep, ai, i)


def _bins_body(a_ref, ct_ref, a2_ref, c2_ref, out_ref):
    k = ct_ref.shape[1]
    a = a_ref[...]
    a2 = a2_ref[...]
    m = a.shape[0]
    lane = lax.broadcasted_iota(jnp.int32, (m, SL), 1).astype(jnp.float32)
    inf = jnp.full((m, SL), jnp.inf, jnp.float32)
    acc_v, acc_s = inf, jnp.zeros((m, SL), jnp.float32)
    wins = []

    def collapse(av, asl):
        v = jnp.min(av, axis=1, keepdims=True)
        kk = asl * float(SL) + lane
        i = jnp.min(jnp.where(av == v, kk, jnp.float32(k)),
                    axis=1, keepdims=True)
        return v, i

    for c in range(k // TK):
        lo = c * TK
        p = lax.dot_general(a, ct_ref[:, lo:lo + TK], (((1,), (0,)), ((), ())),
                            preferred_element_type=jnp.float32)
        d2 = a2 - 2.0 * p + c2_ref[:, lo:lo + TK]
        dist = jnp.sqrt(jnp.maximum(d2, 0.0))
        for t in range(TK // SL):
            s = c * (TK // SL) + t
            d = dist[:, t * SL:(t + 1) * SL]
            bnd = [w for w in (1, 2) if s * SL < w * WIN < (s + 1) * SL]
            if bnd:
                off = bnd[0] * WIN - s * SL
                head = jnp.where(lane < off, d, inf)
                upd = head < acc_v
                acc_v = jnp.where(upd, head, acc_v)
                acc_s = jnp.where(upd, jnp.float32(s), acc_s)
                wins.append(collapse(acc_v, acc_s))
                acc_v, acc_s = inf, jnp.zeros((m, SL), jnp.float32)
                d = jnp.where(lane < off, inf, d)
            upd = d < acc_v
            acc_v = jnp.where(upd, d, acc_v)
            acc_s = jnp.where(upd, jnp.float32(s), acc_s)
    wins.append(collapse(acc_v, acc_s))
    av, ai = wins[0]
    for w in (1, 2):
        av = av.astype(jnp.bfloat16).astype(jnp.float32)
        av, ai = _merge((av, ai), wins[w][0], wins[w][1])
    out_ref[...] = ai.astype(jnp.int32)


def _compute_bins(a_bf, ct_bf, a2, c2):
    n, d = a_bf.shape
    k = ct_bf.shape[1]
    return pl.pallas_call(
        _bins_body,
        grid=(n // TM,),
        in_specs=[
            pl.BlockSpec((TM, d), lambda i: (i, 0)),
            pl.BlockSpec((d, k), lambda i: (0, 0)),
            pl.BlockSpec((TM, 1), lambda i: (i, 0)),
            pl.BlockSpec((1, k), lambda i: (0, 0)),
        ],
        out_specs=pl.BlockSpec((TM, 1), lambda i: (i, 0)),
        out_shape=jax.ShapeDtypeStruct((n, 1), jnp.int32),
    )(a_bf, ct_bf, a2, c2)


def _residual_body(n, cent_hbm, bins_hbm, act_hbm, out_hbm,
                   idx_v, rows_v, act_v, sem):
    wid = lax.axis_index("s") * NC + lax.axis_index("c")
    per_w = n // (NC * NS)
    d = act_v.shape[1]

    def chunk(t, carry):
        base = wid * per_w + t * CH
        pltpu.sync_copy(bins_hbm.at[pl.ds(base, CH)], idx_v)
        gather = pltpu.async_copy(cent_hbm.at[idx_v], rows_v, sem)
        pltpu.sync_copy(act_hbm.at[pl.ds(base, CH), :], act_v)
        gather.wait()

        nv = d // 16

        def sub(i, c):
            r = (4 * i) // nv
            o = ((4 * i) % nv) * 16
            for u in range(4):
                act_v[r, pl.ds(o + 16 * u, 16)] = (
                    act_v[r, pl.ds(o + 16 * u, 16)]
                    - rows_v[r, pl.ds(o + 16 * u, 16)])
            return c

        lax.fori_loop(0, CH * nv // 4, sub, 0)
        pltpu.sync_copy(act_v, out_hbm.at[pl.ds(base, CH), :])
        return carry

    lax.fori_loop(0, per_w // CH, chunk, 0)


def _compute_residual(centroids, bins, flat):
    n, d = flat.shape
    mesh = plsc.VectorSubcoreMesh(core_axis_name="c", subcore_axis_name="s")
    fn = functools.partial(
        pl.kernel,
        out_type=jax.ShapeDtypeStruct((n, d), jnp.float32),
        mesh=mesh,
        scratch_types=[
            pltpu.VMEM((CH,), jnp.int32),
            pltpu.VMEM((CH, d), jnp.float32),
            pltpu.VMEM((CH, d), jnp.float32),
            pltpu.SemaphoreType.DMA,
        ],
    )(functools.partial(_residual_body, n))
    return fn(centroids, bins, flat)


def kernel(action, centroids):
    b, t, d = action.shape
    k = centroids.shape[0]
    flat = action.reshape(b * t, d)
    a2 = jnp.sum(flat * flat, axis=1)
    c2 = jnp.sum(centroids * centroids, axis=1)
    a_bf = flat.astype(jnp.bfloat16)
    ct_bf = centroids.T.astype(jnp.bfloat16)
    bins2d = _compute_bins(a_bf, ct_bf, a2.reshape(b * t, 1),
                           c2.reshape(1, k))
    residual = _compute_residual(centroids, bins2d.reshape(b * t), flat)
    return (bins2d.reshape(b, t, 1).astype(jnp.int64),
            residual.reshape(b, t, d))


# speedup vs baseline: 1.0968x; 1.0466x over previous
import functools

import jax
import jax.numpy as jnp
from jax import lax
from jax.experimental import pallas as pl
from jax.experimental.pallas import tpu as pltpu
from jax.experimental.pallas import tpu_sc as plsc

TM = 128
TK = 512
SL = 128
WIN = 2736
NC = 2
NS = 16
CH = 64


def _merge(acc, v, i):
    av, ai = acc
    keep = (av < v) | ((av == v) & (ai <= i))
    return jnp.where(keep, av, v), jnp.where(keep, ai, i)


def _bins_body(a_ref, ct_ref, a2_ref, c2_ref, out_ref):
    k = ct_ref.shape[1]
    a = a_ref[...].astype(jnp.bfloat16)
    a2 = a2_ref[...]
    m = a.shape[0]
    lane = lax.broadcasted_iota(jnp.int32, (m, SL), 1).astype(jnp.float32)
    inf = jnp.full((m, SL), jnp.inf, jnp.float32)
    acc_v, acc_s = inf, jnp.zeros((m, SL), jnp.float32)
    wins = []

    def collapse(av, asl):
        v = jnp.min(av, axis=1, keepdims=True)
        kk = asl * float(SL) + lane
        i = jnp.min(jnp.where(av == v, kk, jnp.float32(k)),
                    axis=1, keepdims=True)
        return v, i

    for c in range(k // TK):
        lo = c * TK
        p = lax.dot_general(a, ct_ref[:, lo:lo + TK], (((1,), (0,)), ((), ())),
                            preferred_element_type=jnp.float32)
        d2 = a2 - 2.0 * p + c2_ref[:, lo:lo + TK]
        dist = jnp.sqrt(jnp.maximum(d2, 0.0))
        for t in range(TK // SL):
            s = c * (TK // SL) + t
            d = dist[:, t * SL:(t + 1) * SL]
            bnd = [w for w in (1, 2) if s * SL < w * WIN < (s + 1) * SL]
            if bnd:
                off = bnd[0] * WIN - s * SL
                head = jnp.where(lane < off, d, inf)
                upd = head < acc_v
                acc_v = jnp.where(upd, head, acc_v)
                acc_s = jnp.where(upd, jnp.float32(s), acc_s)
                wins.append(collapse(acc_v, acc_s))
                acc_v, acc_s = inf, jnp.zeros((m, SL), jnp.float32)
                d = jnp.where(lane < off, inf, d)
            upd = d < acc_v
            acc_v = jnp.where(upd, d, acc_v)
            acc_s = jnp.where(upd, jnp.float32(s), acc_s)
    wins.append(collapse(acc_v, acc_s))
    av, ai = wins[0]
    for w in (1, 2):
        av = av.astype(jnp.bfloat16).astype(jnp.float32)
        av, ai = _merge((av, ai), wins[w][0], wins[w][1])
    out_ref[...] = ai.astype(jnp.int32)


def _compute_bins(a_bf, ct_bf, a2, c2):
    n, d = a_bf.shape
    k = ct_bf.shape[1]
    return pl.pallas_call(
        _bins_body,
        grid=(n // TM,),
        in_specs=[
            pl.BlockSpec((TM, d), lambda i: (i, 0)),
            pl.BlockSpec((d, k), lambda i: (0, 0)),
            pl.BlockSpec((TM, 1), lambda i: (i, 0)),
            pl.BlockSpec((1, k), lambda i: (0, 0)),
        ],
        out_specs=pl.BlockSpec((TM, 1), lambda i: (i, 0)),
        out_shape=jax.ShapeDtypeStruct((n, 1), jnp.int32),
    )(a_bf, ct_bf, a2, c2)


def _residual_body(n, cent_hbm, bins_hbm, act_hbm, out_hbm,
                   idx_v, rows_v, act_v, sem):
    wid = lax.axis_index("s") * NC + lax.axis_index("c")
    per_w = n // (NC * NS)
    d = act_v.shape[1]

    def chunk(t, carry):
        base = wid * per_w + t * CH
        pltpu.sync_copy(bins_hbm.at[pl.ds(base, CH)], idx_v)
        gather = pltpu.async_copy(cent_hbm.at[idx_v], rows_v, sem)
        pltpu.sync_copy(act_hbm.at[pl.ds(base, CH), :], act_v)
        gather.wait()

        nv = d // 16

        def sub(i, c):
            r = (4 * i) // nv
            o = ((4 * i) % nv) * 16
            for u in range(4):
                act_v[r, pl.ds(o + 16 * u, 16)] = (
                    act_v[r, pl.ds(o + 16 * u, 16)]
                    - rows_v[r, pl.ds(o + 16 * u, 16)])
            return c

        lax.fori_loop(0, CH * nv // 4, sub, 0)
        pltpu.sync_copy(act_v, out_hbm.at[pl.ds(base, CH), :])
        return carry

    lax.fori_loop(0, per_w // CH, chunk, 0)


def _compute_residual(centroids, bins, flat):
    n, d = flat.shape
    mesh = plsc.VectorSubcoreMesh(core_axis_name="c", subcore_axis_name="s")
    fn = functools.partial(
        pl.kernel,
        out_type=jax.ShapeDtypeStruct((n, d), jnp.float32),
        mesh=mesh,
        scratch_types=[
            pltpu.VMEM((CH,), jnp.int32),
            pltpu.VMEM((CH, d), jnp.float32),
            pltpu.VMEM((CH, d), jnp.float32),
            pltpu.SemaphoreType.DMA,
        ],
    )(functools.partial(_residual_body, n))
    return fn(centroids, bins, flat)


def kernel(action, centroids):
    b, t, d = action.shape
    k = centroids.shape[0]
    n = b * t
    flat = action.reshape(n, d)
    a2 = jnp.sum(flat * flat, axis=1).reshape(n, 1)
    c2 = jnp.sum(centroids * centroids, axis=1).reshape(1, k)
    ct_bf = centroids.T.astype(jnp.bfloat16)
    h = n // 2
    bins0 = _compute_bins(flat[:h], ct_bf, a2[:h], c2)
    bins1 = _compute_bins(flat[h:], ct_bf, a2[h:], c2)
    res0 = _compute_residual(centroids, bins0.reshape(h), flat[:h])
    res1 = _compute_residual(centroids, bins1.reshape(h), flat[h:])
    bins2d = jnp.concatenate([bins0, bins1], axis=0)
    residual = jnp.concatenate([res0, res1], axis=0)
    return (bins2d.reshape(b, t, 1).astype(jnp.int64),
            residual.reshape(b, t, d))
